# Initial kernel scaffold; baseline (speedup 1.0000x reference)
import jax
import jax.numpy as jnp
from jax.experimental import pallas as pl


def kernel(features, W1, b1, g1, bt1, W2, b2, g2, bt2):
    raise NotImplementedError("write your pallas kernel here")


# speedup vs baseline: 9.5878x; 9.5878x over previous
import functools

import jax
import jax.numpy as jnp
from jax import lax
from jax.experimental import pallas as pl
from jax.experimental.pallas import tpu as pltpu
from jax.experimental.pallas import tpu_sc as plsc

TN = 256
KNB = 9
_NC, _NS = 2, 16
_NW = _NC * _NS
_CH = 128


def _k12_body(n_pts, xT_ref, x_ref, Wm_ref, gidx_ref, zt_ref, ax_ref):
    b = pl.program_id(0)
    xt = xT_ref[0]
    xb = x_ref[0]
    G = jnp.dot(xt, xb, preferred_element_type=jnp.float32)
    xxr = jnp.sum(xt * xt, axis=1, keepdims=True)
    xxc = jnp.sum(xb * xb, axis=0, keepdims=True)
    S = (2.0 * G - xxr) - xxc
    cols = lax.broadcasted_iota(jnp.int32, S.shape, 1)
    vals = S
    idxs = []
    for _ in range(KNB):
        m = jnp.max(vals, axis=1, keepdims=True)
        eq = vals == m
        am = jnp.min(jnp.where(eq, cols, n_pts), axis=1, keepdims=True)
        idxs.append(am)
        vals = jnp.where(cols == am, -jnp.inf, vals)
    base = b * 3 * n_pts
    for t in range(3):
        for j in range(3):
            gidx_ref[t, 0, j, :] = idxs[3 * j + t][:, 0] + (base + t * n_pts)
    for t in range(3):
        zt_ref[0, t] = jnp.dot(xt, Wm_ref[1 + t],
                               preferred_element_type=jnp.float32)
    ax_ref[0] = jnp.dot(xt, Wm_ref[0], preferred_element_type=jnp.float32)


def _sc_gather(table, gidx_flat):
    rt, cdim = gidx_flat.shape[0], table.shape[1]
    rpw = rt // _NW
    mesh = plsc.VectorSubcoreMesh(core_axis_name="c", subcore_axis_name="s")

    @functools.partial(
        pl.kernel, mesh=mesh,
        out_type=jax.ShapeDtypeStruct((rt, cdim), jnp.float32),
        scratch_types=[
            pltpu.VMEM((_CH,), jnp.int32),
            pltpu.VMEM((_CH, cdim), jnp.float32),
            pltpu.SemaphoreType.DMA,
        ],
    )
    def k(table_hbm, idx_hbm, out_hbm, idx_v, rows_v, sem):
        wid = lax.axis_index("s") * _NC + lax.axis_index("c")
        base = wid * rpw

        def body(c, carry):
            r0 = base + c * _CH
            pltpu.sync_copy(idx_hbm.at[pl.ds(r0, _CH)], idx_v)
            pltpu.async_copy(table_hbm.at[idx_v], rows_v, sem).wait()
            pltpu.sync_copy(rows_v, out_hbm.at[pl.ds(r0, _CH)])
            return carry

        lax.fori_loop(0, rpw // _CH, body, 0)

    return k(table, gidx_flat)


def _k4_body(cdim, gat_ref, ax_ref, b1_ref, y1_ref, st_ref):
    b = pl.program_id(0)
    i = pl.program_id(1)
    axv = ax_ref[0] + b1_ref[...][None, :]
    parts = []
    for j in range(3):
        s = gat_ref[0, 0, j] + gat_ref[1, 0, j] + gat_ref[2, 0, j]
        parts.append(axv - s)
    y1cat = jnp.concatenate(parts, axis=1)
    y1_ref[0] = y1cat

    @pl.when((b == 0) & (i == 0))
    def _():
        st_ref[...] = jnp.zeros_like(st_ref)

    st_ref[0, :] += jnp.sum(y1cat, axis=0)
    st_ref[1, :] += jnp.sum(y1cat * y1cat, axis=0)


def _k5_body(cdim, cnt, y1_ref, st_ref, W2_ref, g1_ref, bt1_ref, b2_ref,
             y2_ref, st2_ref):
    b = pl.program_id(0)
    i = pl.program_id(1)
    s0 = st_ref[0, :]
    s1 = st_ref[1, :]
    m = (s0[:cdim] + s0[cdim:2 * cdim] + s0[2 * cdim:]) / cnt
    v = (s1[:cdim] + s1[cdim:2 * cdim] + s1[2 * cdim:]) / cnt - m * m
    scale = g1_ref[...] * lax.rsqrt(v + 1e-5)
    shift = bt1_ref[...] - m * scale
    sc3 = jnp.concatenate([scale, scale, scale])
    sh3 = jnp.concatenate([shift, shift, shift])
    h1 = jnp.maximum(y1_ref[0] * sc3[None, :] + sh3[None, :], 0.0)
    y2 = jnp.dot(h1, W2_ref[...], preferred_element_type=jnp.float32)
    y2 = y2 + b2_ref[...][None, :]
    y2_ref[0] = y2

    @pl.when((b == 0) & (i == 0))
    def _():
        st2_ref[...] = jnp.zeros_like(st2_ref)

    st2_ref[0, :] += jnp.sum(y2, axis=0)
    st2_ref[1, :] += jnp.sum(y2 * y2, axis=0)


def _k6_body(cnt2, y2_ref, st2_ref, g2_ref, bt2_ref, o_ref):
    s0 = st2_ref[0, :]
    s1 = st2_ref[1, :]
    m = s0 / cnt2
    v = s1 / cnt2 - m * m
    scale = g2_ref[...] * lax.rsqrt(v + 1e-5)
    shift = bt2_ref[...] - m * scale
    h = jnp.maximum(y2_ref[0] * scale[None, :] + shift[None, :], 0.0)
    o_ref[0] = h.T


def kernel(features, W1, b1, g1, bt1, W2, b2, g2, bt2):
    bsz, cdim, n_pts, _ = features.shape
    nt = n_pts // TN
    x = features.reshape(bsz, cdim, n_pts)
    xT = jnp.transpose(x, (0, 2, 1))

    W1s = W1[:, :, 0, :]
    W1a, W1b = W1s[:, :cdim, :], W1s[:, cdim:, :]
    A_m = jnp.sum(W1a + W1b, axis=2).T
    Wm = jnp.stack([A_m, W1b[:, :, 0].T, W1b[:, :, 1].T, W1b[:, :, 2].T])
    W2cat = W2[:, :, 0, :].transpose(2, 1, 0).reshape(3 * cdim, cdim)

    gidx, zt, ax = pl.pallas_call(
        functools.partial(_k12_body, n_pts),
        grid=(bsz, nt),
        in_specs=[
            pl.BlockSpec((1, TN, cdim), lambda b, i: (b, i, 0)),
            pl.BlockSpec((1, cdim, n_pts), lambda b, i: (b, 0, 0)),
            pl.BlockSpec((4, cdim, cdim), lambda b, i: (0, 0, 0)),
        ],
        out_specs=[
            pl.BlockSpec((3, 1, 3, TN), lambda b, i: (0, b, 0, i)),
            pl.BlockSpec((1, 3, TN, cdim), lambda b, i: (b, 0, i, 0)),
            pl.BlockSpec((1, TN, cdim), lambda b, i: (b, i, 0)),
        ],
        out_shape=[
            jax.ShapeDtypeStruct((3, bsz, 3, n_pts), jnp.int32),
            jax.ShapeDtypeStruct((bsz, 3, n_pts, cdim), jnp.float32),
            jax.ShapeDtypeStruct((bsz, n_pts, cdim), jnp.float32),
        ],
    )(xT, x, Wm)

    table = zt.reshape(bsz * 3 * n_pts, cdim)
    gat_flat = _sc_gather(table, gidx.reshape(-1))
    gat = gat_flat.reshape(3, bsz, 3, n_pts, cdim)

    y1, st1 = pl.pallas_call(
        functools.partial(_k4_body, cdim),
        grid=(bsz, nt),
        in_specs=[
            pl.BlockSpec((3, 1, 3, TN, cdim), lambda b, i: (0, b, 0, i, 0)),
            pl.BlockSpec((1, TN, cdim), lambda b, i: (b, i, 0)),
            pl.BlockSpec((cdim,), lambda b, i: (0,)),
        ],
        out_specs=[
            pl.BlockSpec((1, TN, 3 * cdim), lambda b, i: (b, i, 0)),
            pl.BlockSpec((2, 3 * cdim), lambda b, i: (0, 0)),
        ],
        out_shape=[
            jax.ShapeDtypeStruct((bsz, n_pts, 3 * cdim), jnp.float32),
            jax.ShapeDtypeStruct((2, 3 * cdim), jnp.float32),
        ],
    )(gat, ax, b1)

    y2, st2 = pl.pallas_call(
        functools.partial(_k5_body, cdim, float(bsz * n_pts * 3)),
        grid=(bsz, nt),
        in_specs=[
            pl.BlockSpec((1, TN, 3 * cdim), lambda b, i: (b, i, 0)),
            pl.BlockSpec((2, 3 * cdim), lambda b, i: (0, 0)),
            pl.BlockSpec((3 * cdim, cdim), lambda b, i: (0, 0)),
            pl.BlockSpec((cdim,), lambda b, i: (0,)),
            pl.BlockSpec((cdim,), lambda b, i: (0,)),
            pl.BlockSpec((cdim,), lambda b, i: (0,)),
        ],
        out_specs=[
            pl.BlockSpec((1, TN, cdim), lambda b, i: (b, i, 0)),
            pl.BlockSpec((2, cdim), lambda b, i: (0, 0)),
        ],
        out_shape=[
            jax.ShapeDtypeStruct((bsz, n_pts, cdim), jnp.float32),
            jax.ShapeDtypeStruct((2, cdim), jnp.float32),
        ],
    )(y1, st1, W2cat, g1, bt1, b2)

    out = pl.pallas_call(
        functools.partial(_k6_body, float(bsz * n_pts)),
        grid=(bsz, nt),
        in_specs=[
            pl.BlockSpec((1, TN, cdim), lambda b, i: (b, i, 0)),
            pl.BlockSpec((2, cdim), lambda b, i: (0, 0)),
            pl.BlockSpec((cdim,), lambda b, i: (0,)),
            pl.BlockSpec((cdim,), lambda b, i: (0,)),
        ],
        out_specs=pl.BlockSpec((1, cdim, TN), lambda b, i: (b, 0, i)),
        out_shape=jax.ShapeDtypeStruct((bsz, cdim, n_pts), jnp.float32),
    )(y2, st2, g2, bt2)

    return out[:, :, :, None]


# speedup vs baseline: 11.4290x; 1.1920x over previous
import functools

import jax
import jax.numpy as jnp
from jax import lax
from jax.experimental import pallas as pl
from jax.experimental.pallas import tpu as pltpu
from jax.experimental.pallas import tpu_sc as plsc

TN = 256
KNB = 9
_NC, _NS = 2, 16
_NW = _NC * _NS
_CH = 128


def _k12_body(n_pts, xT_ref, x_ref, Wm_ref, gidx_ref, zt_ref, ax_ref):
    b = pl.program_id(0)
    xt = xT_ref[0]
    xb = x_ref[0]
    G = jnp.dot(xt, xb, preferred_element_type=jnp.float32)
    xxr = jnp.sum(xt * xt, axis=1, keepdims=True)
    xxc = jnp.sum(xb * xb, axis=0, keepdims=True)
    S = (2.0 * G - xxr) - xxc
    cols = lax.broadcasted_iota(jnp.int32, S.shape, 1)
    vals = S
    idxs = []
    for _ in range(KNB):
        m = jnp.max(vals, axis=1, keepdims=True)
        eq = vals == m
        am = jnp.min(jnp.where(eq, cols, n_pts), axis=1, keepdims=True)
        idxs.append(am)
        vals = jnp.where(eq, -jnp.inf, vals)
    idxs += [jnp.zeros_like(idxs[0])] * (16 - KNB)
    idx_mat = jnp.concatenate(idxs, axis=1)
    k16 = lax.broadcasted_iota(jnp.int32, (1, 16), 1)
    off = (b * 3 + k16 % 3) * n_pts
    gidx_ref[0] = idx_mat + off
    for t in range(3):
        zt_ref[0, t] = jnp.dot(xt, Wm_ref[1 + t],
                               preferred_element_type=jnp.float32)
    ax_ref[0] = jnp.dot(xt, Wm_ref[0], preferred_element_type=jnp.float32)


def _sc_gather(table, gidx_flat):
    rt, cdim = gidx_flat.shape[0], table.shape[1]
    rpw = rt // _NW
    mesh = plsc.VectorSubcoreMesh(core_axis_name="c", subcore_axis_name="s")

    @functools.partial(
        pl.kernel, mesh=mesh,
        out_type=jax.ShapeDtypeStruct((rt, cdim), jnp.float32),
        scratch_types=[
            pltpu.VMEM((_CH,), jnp.int32),
            pltpu.VMEM((_CH, cdim), jnp.float32),
            pltpu.SemaphoreType.DMA,
        ],
    )
    def k(table_hbm, idx_hbm, out_hbm, idx_v, rows_v, sem):
        wid = lax.axis_index("s") * _NC + lax.axis_index("c")
        base = wid * rpw

        def body(c, carry):
            r0 = base + c * _CH
            pltpu.sync_copy(idx_hbm.at[pl.ds(r0, _CH)], idx_v)
            pltpu.async_copy(table_hbm.at[idx_v], rows_v, sem).wait()
            pltpu.sync_copy(rows_v, out_hbm.at[pl.ds(r0, _CH)])
            return carry

        lax.fori_loop(0, rpw // _CH, body, 0)

    return k(table, gidx_flat)


def _k4_body(cdim, gat_ref, ax_ref, b1_ref, y1_ref, st_ref):
    b = pl.program_id(0)
    i = pl.program_id(1)
    axv = ax_ref[0] + b1_ref[...][None, :]
    parts = []
    for j in range(3):
        s = gat_ref[0, 0, j] + gat_ref[1, 0, j] + gat_ref[2, 0, j]
        parts.append(axv - s)
    y1cat = jnp.concatenate(parts, axis=1)
    y1_ref[0] = y1cat

    @pl.when((b == 0) & (i == 0))
    def _():
        st_ref[...] = jnp.zeros_like(st_ref)

    st_ref[0, :] += jnp.sum(y1cat, axis=0)
    st_ref[1, :] += jnp.sum(y1cat * y1cat, axis=0)


def _k5_body(cdim, cnt, y1_ref, st_ref, W2_ref, g1_ref, bt1_ref, b2_ref,
             y2_ref, st2_ref):
    b = pl.program_id(0)
    i = pl.program_id(1)
    s0 = st_ref[0, :]
    s1 = st_ref[1, :]
    m = (s0[:cdim] + s0[cdim:2 * cdim] + s0[2 * cdim:]) / cnt
    v = (s1[:cdim] + s1[cdim:2 * cdim] + s1[2 * cdim:]) / cnt - m * m
    scale = g1_ref[...] * lax.rsqrt(v + 1e-5)
    shift = bt1_ref[...] - m * scale
    sc3 = jnp.concatenate([scale, scale, scale])
    sh3 = jnp.concatenate([shift, shift, shift])
    h1 = jnp.maximum(y1_ref[0] * sc3[None, :] + sh3[None, :], 0.0)
    y2 = jnp.dot(h1, W2_ref[...], preferred_element_type=jnp.float32)
    y2 = y2 + b2_ref[...][None, :]
    y2_ref[0] = y2

    @pl.when((b == 0) & (i == 0))
    def _():
        st2_ref[...] = jnp.zeros_like(st2_ref)

    st2_ref[0, :] += jnp.sum(y2, axis=0)
    st2_ref[1, :] += jnp.sum(y2 * y2, axis=0)


def _k6_body(cnt2, y2_ref, st2_ref, g2_ref, bt2_ref, o_ref):
    s0 = st2_ref[0, :]
    s1 = st2_ref[1, :]
    m = s0 / cnt2
    v = s1 / cnt2 - m * m
    scale = g2_ref[...] * lax.rsqrt(v + 1e-5)
    shift = bt2_ref[...] - m * scale
    h = jnp.maximum(y2_ref[0] * scale[None, :] + shift[None, :], 0.0)
    o_ref[0] = h.T


def kernel(features, W1, b1, g1, bt1, W2, b2, g2, bt2):
    bsz, cdim, n_pts, _ = features.shape
    nt = n_pts // TN
    x = features.reshape(bsz, cdim, n_pts)
    xT = jnp.transpose(x, (0, 2, 1))

    W1s = W1[:, :, 0, :]
    W1a, W1b = W1s[:, :cdim, :], W1s[:, cdim:, :]
    A_m = jnp.sum(W1a + W1b, axis=2).T
    Wm = jnp.stack([A_m, W1b[:, :, 0].T, W1b[:, :, 1].T, W1b[:, :, 2].T])
    W2cat = W2[:, :, 0, :].transpose(2, 1, 0).reshape(3 * cdim, cdim)

    gidx, zt, ax = pl.pallas_call(
        functools.partial(_k12_body, n_pts),
        grid=(bsz, nt),
        in_specs=[
            pl.BlockSpec((1, TN, cdim), lambda b, i: (b, i, 0)),
            pl.BlockSpec((1, cdim, n_pts), lambda b, i: (b, 0, 0)),
            pl.BlockSpec((4, cdim, cdim), lambda b, i: (0, 0, 0)),
        ],
        out_specs=[
            pl.BlockSpec((1, TN, 16), lambda b, i: (b, i, 0)),
            pl.BlockSpec((1, 3, TN, cdim), lambda b, i: (b, 0, i, 0)),
            pl.BlockSpec((1, TN, cdim), lambda b, i: (b, i, 0)),
        ],
        out_shape=[
            jax.ShapeDtypeStruct((bsz, n_pts, 16), jnp.int32),
            jax.ShapeDtypeStruct((bsz, 3, n_pts, cdim), jnp.float32),
            jax.ShapeDtypeStruct((bsz, n_pts, cdim), jnp.float32),
        ],
    )(xT, x, Wm)

    table = zt.reshape(bsz * 3 * n_pts, cdim)
    gidx_flat = (gidx[:, :, :KNB].reshape(bsz, n_pts, 3, 3)
                 .transpose(3, 0, 2, 1).reshape(-1))
    gat_flat = _sc_gather(table, gidx_flat)
    gat = gat_flat.reshape(3, bsz, 3, n_pts, cdim)

    y1, st1 = pl.pallas_call(
        functools.partial(_k4_body, cdim),
        grid=(bsz, nt),
        in_specs=[
            pl.BlockSpec((3, 1, 3, TN, cdim), lambda b, i: (0, b, 0, i, 0)),
            pl.BlockSpec((1, TN, cdim), lambda b, i: (b, i, 0)),
            pl.BlockSpec((cdim,), lambda b, i: (0,)),
        ],
        out_specs=[
            pl.BlockSpec((1, TN, 3 * cdim), lambda b, i: (b, i, 0)),
            pl.BlockSpec((2, 3 * cdim), lambda b, i: (0, 0)),
        ],
        out_shape=[
            jax.ShapeDtypeStruct((bsz, n_pts, 3 * cdim), jnp.float32),
            jax.ShapeDtypeStruct((2, 3 * cdim), jnp.float32),
        ],
    )(gat, ax, b1)

    y2, st2 = pl.pallas_call(
        functools.partial(_k5_body, cdim, float(bsz * n_pts * 3)),
        grid=(bsz, nt),
        in_specs=[
            pl.BlockSpec((1, TN, 3 * cdim), lambda b, i: (b, i, 0)),
            pl.BlockSpec((2, 3 * cdim), lambda b, i: (0, 0)),
            pl.BlockSpec((3 * cdim, cdim), lambda b, i: (0, 0)),
            pl.BlockSpec((cdim,), lambda b, i: (0,)),
            pl.BlockSpec((cdim,), lambda b, i: (0,)),
            pl.BlockSpec((cdim,), lambda b, i: (0,)),
        ],
        out_specs=[
            pl.BlockSpec((1, TN, cdim), lambda b, i: (b, i, 0)),
            pl.BlockSpec((2, cdim), lambda b, i: (0, 0)),
        ],
        out_shape=[
            jax.ShapeDtypeStruct((bsz, n_pts, cdim), jnp.float32),
            jax.ShapeDtypeStruct((2, cdim), jnp.float32),
        ],
    )(y1, st1, W2cat, g1, bt1, b2)

    out = pl.pallas_call(
        functools.partial(_k6_body, float(bsz * n_pts)),
        grid=(bsz, nt),
        in_specs=[
            pl.BlockSpec((1, TN, cdim), lambda b, i: (b, i, 0)),
            pl.BlockSpec((2, cdim), lambda b, i: (0, 0)),
            pl.BlockSpec((cdim,), lambda b, i: (0,)),
            pl.BlockSpec((cdim,), lambda b, i: (0,)),
        ],
        out_specs=pl.BlockSpec((1, cdim, TN), lambda b, i: (b, 0, i)),
        out_shape=jax.ShapeDtypeStruct((bsz, cdim, n_pts), jnp.float32),
    )(y2, st2, g2, bt2)

    return out[:, :, :, None]


# speedup vs baseline: 12.0485x; 1.0542x over previous
import functools

import jax
import jax.numpy as jnp
from jax import lax
from jax.experimental import pallas as pl
from jax.experimental.pallas import tpu as pltpu
from jax.experimental.pallas import tpu_sc as plsc

TN = 256
KNB = 9
_NC, _NS = 2, 16
_NW = _NC * _NS
_CH = 128


def _k12_body(n_pts, xT_ref, x_ref, Wm_ref, gidx_ref, zt_ref, ax_ref):
    b = pl.program_id(0)
    xt = xT_ref[0]
    xb = x_ref[0]
    G = jnp.dot(xt, xb, preferred_element_type=jnp.float32)
    xxr = jnp.sum(xt * xt, axis=1, keepdims=True)
    xxc = jnp.sum(xb * xb, axis=0, keepdims=True)
    S = (2.0 * G - xxr) - xxc
    cols = lax.broadcasted_iota(jnp.int32, S.shape, 1)
    vals = S
    idxs = []
    for _ in range(KNB):
        m = jnp.max(vals, axis=1, keepdims=True)
        eq = vals == m
        am = jnp.min(jnp.where(eq, cols, n_pts), axis=1, keepdims=True)
        idxs.append(am)
        vals = jnp.where(eq, -jnp.inf, vals)
    idxs += [jnp.zeros_like(idxs[0])] * (16 - KNB)
    idx_mat = jnp.concatenate(idxs, axis=1)
    k16 = lax.broadcasted_iota(jnp.int32, (1, 16), 1)
    off = (b * 3 + k16 % 3) * n_pts
    gidx_ref[0] = idx_mat + off
    for t in range(3):
        zt_ref[0, t] = jnp.dot(xt, Wm_ref[1 + t],
                               preferred_element_type=jnp.float32)
    ax_ref[0] = jnp.dot(xt, Wm_ref[0], preferred_element_type=jnp.float32)


def _sc_gather(table, gidx_flat):
    rt, cdim = gidx_flat.shape[0], table.shape[1]
    rpw = rt // _NW
    mesh = plsc.VectorSubcoreMesh(core_axis_name="c", subcore_axis_name="s")

    @functools.partial(
        pl.kernel, mesh=mesh,
        out_type=jax.ShapeDtypeStruct((rt, cdim), jnp.float32),
        scratch_types=[
            pltpu.VMEM((_CH,), jnp.int32),
            pltpu.VMEM((_CH, cdim), jnp.float32),
            pltpu.SemaphoreType.DMA,
        ],
    )
    def k(table_hbm, idx_hbm, out_hbm, idx_v, rows_v, sem):
        wid = lax.axis_index("s") * _NC + lax.axis_index("c")
        base = wid * rpw

        def body(c, carry):
            r0 = base + c * _CH
            pltpu.sync_copy(idx_hbm.at[pl.ds(r0, _CH)], idx_v)
            pltpu.async_copy(table_hbm.at[idx_v], rows_v, sem).wait()
            pltpu.sync_copy(rows_v, out_hbm.at[pl.ds(r0, _CH)])
            return carry

        lax.fori_loop(0, rpw // _CH, body, 0)

    return k(table, gidx_flat)


def _k4_body(cdim, gat_ref, ax_ref, b1_ref, y1_ref, st_ref):
    b = pl.program_id(0)
    i = pl.program_id(1)
    axv = ax_ref[0] + b1_ref[...][None, :]
    parts = []
    for j in range(3):
        s = gat_ref[0, 0, j] + gat_ref[1, 0, j] + gat_ref[2, 0, j]
        parts.append(axv - s)
    y1cat = jnp.concatenate(parts, axis=1)
    y1_ref[0] = y1cat

    @pl.when((b == 0) & (i == 0))
    def _():
        st_ref[...] = jnp.zeros_like(st_ref)

    st_ref[0, :] += jnp.sum(y1cat, axis=0)
    st_ref[1, :] += jnp.sum(y1cat * y1cat, axis=0)


def _k5_body(cdim, cnt, y1_ref, st_ref, W2_ref, g1_ref, bt1_ref, b2_ref,
             y2_ref, st2_ref):
    b = pl.program_id(0)
    i = pl.program_id(1)
    s0 = st_ref[0, :]
    s1 = st_ref[1, :]
    m = (s0[:cdim] + s0[cdim:2 * cdim] + s0[2 * cdim:]) / cnt
    v = (s1[:cdim] + s1[cdim:2 * cdim] + s1[2 * cdim:]) / cnt - m * m
    scale = g1_ref[...] * lax.rsqrt(v + 1e-5)
    shift = bt1_ref[...] - m * scale
    sc3 = jnp.concatenate([scale, scale, scale])
    sh3 = jnp.concatenate([shift, shift, shift])
    h1 = jnp.maximum(y1_ref[0] * sc3[None, :] + sh3[None, :], 0.0)
    y2 = jnp.dot(h1, W2_ref[...], preferred_element_type=jnp.float32)
    y2 = y2 + b2_ref[...][None, :]
    y2_ref[0] = y2

    @pl.when((b == 0) & (i == 0))
    def _():
        st2_ref[...] = jnp.zeros_like(st2_ref)

    st2_ref[0, :] += jnp.sum(y2, axis=0)
    st2_ref[1, :] += jnp.sum(y2 * y2, axis=0)


def _k6_body(cnt2, y2_ref, st2_ref, g2_ref, bt2_ref, o_ref):
    s0 = st2_ref[0, :]
    s1 = st2_ref[1, :]
    m = s0 / cnt2
    v = s1 / cnt2 - m * m
    scale = g2_ref[...] * lax.rsqrt(v + 1e-5)
    shift = bt2_ref[...] - m * scale
    h = jnp.maximum(y2_ref[0] * scale[None, :] + shift[None, :], 0.0)
    o_ref[0] = h.T


NG = 2


def kernel(features, W1, b1, g1, bt1, W2, b2, g2, bt2):
    bsz, cdim, n_pts, _ = features.shape
    nt = n_pts // TN
    gb = bsz // NG
    x = features.reshape(bsz, cdim, n_pts)
    xT = jnp.transpose(x, (0, 2, 1))

    W1s = W1[:, :, 0, :]
    W1a, W1b = W1s[:, :cdim, :], W1s[:, cdim:, :]
    A_m = jnp.sum(W1a + W1b, axis=2).T
    Wm = jnp.stack([A_m, W1b[:, :, 0].T, W1b[:, :, 1].T, W1b[:, :, 2].T])
    W2cat = W2[:, :, 0, :].transpose(2, 1, 0).reshape(3 * cdim, cdim)

    gats, axs, y1s, st1s = [], [], [], []
    for g in range(NG):
        gidx, zt, ax = pl.pallas_call(
            functools.partial(_k12_body, n_pts),
            grid=(gb, nt),
            in_specs=[
                pl.BlockSpec((1, TN, cdim), lambda b, i: (b, i, 0)),
                pl.BlockSpec((1, cdim, n_pts), lambda b, i: (b, 0, 0)),
                pl.BlockSpec((4, cdim, cdim), lambda b, i: (0, 0, 0)),
            ],
            out_specs=[
                pl.BlockSpec((1, TN, 16), lambda b, i: (b, i, 0)),
                pl.BlockSpec((1, 3, TN, cdim), lambda b, i: (b, 0, i, 0)),
                pl.BlockSpec((1, TN, cdim), lambda b, i: (b, i, 0)),
            ],
            out_shape=[
                jax.ShapeDtypeStruct((gb, n_pts, 16), jnp.int32),
                jax.ShapeDtypeStruct((gb, 3, n_pts, cdim), jnp.float32),
                jax.ShapeDtypeStruct((gb, n_pts, cdim), jnp.float32),
            ],
        )(xT[g * gb:(g + 1) * gb], x[g * gb:(g + 1) * gb], Wm)

        table = zt.reshape(gb * 3 * n_pts, cdim)
        gidx_flat = (gidx[:, :, :KNB].reshape(gb, n_pts, 3, 3)
                     .transpose(3, 0, 2, 1).reshape(-1))
        gat_flat = _sc_gather(table, gidx_flat)
        gats.append(gat_flat.reshape(3, gb, 3, n_pts, cdim))
        axs.append(ax)

    for g in range(NG):
        y1, st1 = pl.pallas_call(
            functools.partial(_k4_body, cdim),
            grid=(gb, nt),
            in_specs=[
                pl.BlockSpec((3, 1, 3, TN, cdim), lambda b, i: (0, b, 0, i, 0)),
                pl.BlockSpec((1, TN, cdim), lambda b, i: (b, i, 0)),
                pl.BlockSpec((cdim,), lambda b, i: (0,)),
            ],
            out_specs=[
                pl.BlockSpec((1, TN, 3 * cdim), lambda b, i: (b, i, 0)),
                pl.BlockSpec((2, 3 * cdim), lambda b, i: (0, 0)),
            ],
            out_shape=[
                jax.ShapeDtypeStruct((gb, n_pts, 3 * cdim), jnp.float32),
                jax.ShapeDtypeStruct((2, 3 * cdim), jnp.float32),
            ],
        )(gats[g], axs[g], b1)
        y1s.append(y1)
        st1s.append(st1)
    st1 = st1s[0] + st1s[1]

    y2s, st2s = [], []
    for g in range(NG):
        y2, st2 = pl.pallas_call(
            functools.partial(_k5_body, cdim, float(bsz * n_pts * 3)),
            grid=(gb, nt),
            in_specs=[
                pl.BlockSpec((1, TN, 3 * cdim), lambda b, i: (b, i, 0)),
                pl.BlockSpec((2, 3 * cdim), lambda b, i: (0, 0)),
                pl.BlockSpec((3 * cdim, cdim), lambda b, i: (0, 0)),
                pl.BlockSpec((cdim,), lambda b, i: (0,)),
                pl.BlockSpec((cdim,), lambda b, i: (0,)),
                pl.BlockSpec((cdim,), lambda b, i: (0,)),
            ],
            out_specs=[
                pl.BlockSpec((1, TN, cdim), lambda b, i: (b, i, 0)),
                pl.BlockSpec((2, cdim), lambda b, i: (0, 0)),
            ],
            out_shape=[
                jax.ShapeDtypeStruct((gb, n_pts, cdim), jnp.float32),
                jax.ShapeDtypeStruct((2, cdim), jnp.float32),
            ],
        )(y1s[g], st1, W2cat, g1, bt1, b2)
        y2s.append(y2)
        st2s.append(st2)
    st2 = st2s[0] + st2s[1]

    outs = []
    for g in range(NG):
        out = pl.pallas_call(
            functools.partial(_k6_body, float(bsz * n_pts)),
            grid=(gb, nt),
            in_specs=[
                pl.BlockSpec((1, TN, cdim), lambda b, i: (b, i, 0)),
                pl.BlockSpec((2, cdim), lambda b, i: (0, 0)),
                pl.BlockSpec((cdim,), lambda b, i: (0,)),
                pl.BlockSpec((cdim,), lambda b, i: (0,)),
            ],
            out_specs=pl.BlockSpec((1, cdim, TN), lambda b, i: (b, 0, i)),
            out_shape=jax.ShapeDtypeStruct((gb, cdim, n_pts), jnp.float32),
        )(y2s[g], st2, g2, bt2)
        outs.append(out)

    return jnp.concatenate(outs, axis=0)[:, :, :, None]
